# baseline (speedup 1.0000x reference)
import functools

import numpy as np
import jax
import jax.numpy as jnp
from jax import lax
from jax.experimental import pallas as pl
from jax.experimental.pallas import tpu as pltpu
from jax.experimental.pallas import tpu_sc as plsc

jax.config.update('jax_enable_x64', True)

B = 16384
F = 26
NUM_BUCKETS = 10000
N = B * F

_info = plsc.get_sparse_core_info()
NC, NS, L = _info.num_cores, _info.num_subcores, _info.num_lanes
NW = NC * NS
CHUNK = N // NW
assert CHUNK * NW == N and CHUNK % L == 0


def _body(x_hbm, vid_hbm, valid_hbm, x_v, vid_v, valid_v):
    wid = lax.axis_index("s") * jnp.int32(NC) + lax.axis_index("c")
    base = wid * jnp.int32(CHUNK)
    pltpu.sync_copy(x_hbm.at[pl.ds(base, CHUNK)], x_v)

    lane = lax.iota(jnp.int32, L)

    @plsc.parallel_loop(jnp.int32(0), jnp.int32(CHUNK), jnp.int32(L),
                        unroll=4)
    def step(li):
        raw = x_v[pl.ds(li, L)]
        e0 = base + li
        field = lax.rem(e0 + lane, jnp.int32(F))
        id_base = field * jnp.int32(NUM_BUCKETS + 1) + jnp.int32(1)
        r = lax.rem(raw, jnp.int32(NUM_BUCKETS))
        bucket = r + jnp.int32(1)
        ok = raw > 0
        vid = lax.select(ok, id_base + bucket, jnp.zeros((L,), jnp.int32))
        vmask = lax.select(ok, jnp.ones((L,), jnp.int32),
                           jnp.zeros((L,), jnp.int32))
        vid_v[pl.ds(li, L)] = vid
        valid_v[pl.ds(li, L)] = vmask

    pltpu.sync_copy(vid_v, vid_hbm.at[pl.ds(base, CHUNK)])
    pltpu.sync_copy(valid_v, valid_hbm.at[pl.ds(base, CHUNK)])


@jax.jit
def kernel(int_feats):
    x32 = int_feats.astype(jnp.int32).reshape(N)
    run = functools.partial(
        pl.kernel,
        mesh=plsc.VectorSubcoreMesh(core_axis_name="c", subcore_axis_name="s"),
        out_type=[
            jax.ShapeDtypeStruct((N,), jnp.int32),
            jax.ShapeDtypeStruct((N,), jnp.int32),
        ],
        scratch_types=[
            pltpu.VMEM((CHUNK,), jnp.int32),
            pltpu.VMEM((CHUNK,), jnp.int32),
            pltpu.VMEM((CHUNK,), jnp.int32),
        ],
    )(_body)
    vid32, valid32 = run(x32)
    vids = vid32.astype(jnp.int64).reshape(B, F)
    valid = valid32.astype(jnp.bool_).reshape(B, F)
    return vids, valid
